# baseline (speedup 1.0000x reference)
import functools

import jax
import jax.numpy as jnp
from jax import lax
from jax.experimental import pallas as pl
from jax.experimental.pallas import tpu as pltpu
from jax.experimental.pallas import tpu_sc as plsc

_NUM_CORES = 2
_NUM_SUBCORES = 16
_NW = _NUM_CORES * _NUM_SUBCORES

_PACK = 8
_TILE_N = 2048


def _gather_sc(x8, table128):
    batch = x8.shape[0]
    width = table128.shape[1]
    b_per_w = batch // _NW
    mesh = plsc.VectorSubcoreMesh(core_axis_name="c", subcore_axis_name="s")

    @functools.partial(
        pl.kernel,
        mesh=mesh,
        out_type=jax.ShapeDtypeStruct((batch, width), jnp.float32),
        scratch_types=[
            pltpu.VMEM((b_per_w,), jnp.int32),
            pltpu.VMEM((b_per_w, width), jnp.float32),
            pltpu.SemaphoreType.DMA,
        ],
    )
    def gather_kernel(idx_hbm, table_hbm, out_hbm, idx_v, rows_v, sem):
        wid = lax.axis_index("s") * _NUM_CORES + lax.axis_index("c")
        base = wid * b_per_w
        pltpu.sync_copy(idx_hbm.at[pl.ds(base, b_per_w)], idx_v)
        pltpu.async_copy(table_hbm.at[idx_v], rows_v, sem).wait()
        pltpu.sync_copy(rows_v, out_hbm.at[pl.ds(base, b_per_w)])

    return gather_kernel(x8, table128)


def _dense_body(rows_ref, off_ref, w1t_ref, b1_ref, w2_ref, b2_ref, out_ref,
                h_ref):
    @pl.when(pl.program_id(0) == 0)
    def _():
        lane_slot = lax.broadcasted_iota(jnp.int32, rows_ref.shape, 1) // 16
        masked = jnp.where(lane_slot == off_ref[...], rows_ref[...], 0.0)
        h = jnp.dot(masked, w1t_ref[...],
                    preferred_element_type=jnp.float32,
                    precision=lax.Precision.HIGHEST) + b1_ref[...]
        h_ref[...] = jnp.maximum(h, 0.0)

    out_ref[...] = jnp.dot(h_ref[...], w2_ref[...],
                           preferred_element_type=jnp.float32) + b2_ref[...]


def _dense(rows128, offs, W1t, b1, W2, b2):
    batch, width = rows128.shape
    hidden = W1t.shape[1]
    n = W2.shape[1]
    grid = pl.cdiv(n, _TILE_N)
    return pl.pallas_call(
        _dense_body,
        grid=(grid,),
        in_specs=[
            pl.BlockSpec((batch, width), lambda i: (0, 0)),
            pl.BlockSpec((batch, 1), lambda i: (0, 0)),
            pl.BlockSpec((width, hidden), lambda i: (0, 0)),
            pl.BlockSpec((1, hidden), lambda i: (0, 0)),
            pl.BlockSpec((W2.shape[0], _TILE_N), lambda i: (0, i)),
            pl.BlockSpec((1, _TILE_N), lambda i: (0, i)),
        ],
        out_specs=pl.BlockSpec((batch, _TILE_N), lambda i: (0, i)),
        out_shape=jax.ShapeDtypeStruct((batch, n), jnp.float32),
        scratch_shapes=[pltpu.VMEM((batch, hidden), jnp.float32)],
    )(rows128, offs, W1t, b1.reshape(1, hidden), W2, b2.reshape(1, n))


def kernel(x, embedding, W1, b1, W2, b2):
    num_cat, dim = embedding.shape
    table128 = embedding.reshape(num_cat // _PACK, dim * _PACK)
    x = x.astype(jnp.int32)
    rows128 = _gather_sc(x // _PACK, table128)
    offs = (x % _PACK).reshape(x.shape[0], 1)
    W1t = jnp.tile(W1, (_PACK, 1))
    return _dense(rows128, offs, W1t, b1, W2, b2)


# speedup vs baseline: 1.0024x; 1.0024x over previous
import functools

import jax
import jax.numpy as jnp
from jax import lax
from jax.experimental import pallas as pl
from jax.experimental.pallas import tpu as pltpu
from jax.experimental.pallas import tpu_sc as plsc

_NUM_CORES = 2
_NUM_SUBCORES = 16
_NW = _NUM_CORES * _NUM_SUBCORES

_PACK = 8
_TILE_N = 2048


def _gather_sc(x8, table128):
    batch = x8.shape[0]
    width = table128.shape[1]
    b_per_w = batch // _NW
    mesh = plsc.VectorSubcoreMesh(core_axis_name="c", subcore_axis_name="s")

    @functools.partial(
        pl.kernel,
        mesh=mesh,
        out_type=jax.ShapeDtypeStruct((batch, width), jnp.float32),
        scratch_types=[
            pltpu.VMEM((b_per_w,), jnp.int32),
            pltpu.VMEM((b_per_w, width), jnp.float32),
            pltpu.SemaphoreType.DMA,
        ],
    )
    def gather_kernel(idx_hbm, table_hbm, out_hbm, idx_v, rows_v, sem):
        wid = lax.axis_index("s") * _NUM_CORES + lax.axis_index("c")
        base = wid * b_per_w
        pltpu.sync_copy(idx_hbm.at[pl.ds(base, b_per_w)], idx_v)
        pltpu.async_copy(table_hbm.at[idx_v], rows_v, sem).wait()
        pltpu.sync_copy(rows_v, out_hbm.at[pl.ds(base, b_per_w)])

    return gather_kernel(x8, table128)


def _dense_body(rows_ref, off_ref, w1t_ref, b1_ref, w2_ref, b2_ref, out_ref,
                h_ref):
    @pl.when(pl.program_id(0) == 0)
    def _():
        lane_slot = lax.broadcasted_iota(jnp.int32, rows_ref.shape, 1) // 16
        masked = jnp.where(lane_slot == off_ref[...], rows_ref[...], 0.0)
        h = jnp.dot(masked, w1t_ref[...],
                    preferred_element_type=jnp.float32,
                    precision=lax.Precision.HIGHEST) + b1_ref[...]
        h_ref[...] = jnp.maximum(h, 0.0).astype(jnp.bfloat16)

    out_ref[...] = jnp.dot(h_ref[...], w2_ref[...].astype(jnp.bfloat16),
                           preferred_element_type=jnp.float32) + b2_ref[...]


def _dense(rows128, offs, W1t, b1, W2, b2):
    batch, width = rows128.shape
    hidden = W1t.shape[1]
    n = W2.shape[1]
    grid = pl.cdiv(n, _TILE_N)
    return pl.pallas_call(
        _dense_body,
        grid=(grid,),
        in_specs=[
            pl.BlockSpec((batch, width), lambda i: (0, 0)),
            pl.BlockSpec((batch, 1), lambda i: (0, 0)),
            pl.BlockSpec((width, hidden), lambda i: (0, 0)),
            pl.BlockSpec((1, hidden), lambda i: (0, 0)),
            pl.BlockSpec((W2.shape[0], _TILE_N), lambda i: (0, i)),
            pl.BlockSpec((1, _TILE_N), lambda i: (0, i)),
        ],
        out_specs=pl.BlockSpec((batch, _TILE_N), lambda i: (0, i)),
        out_shape=jax.ShapeDtypeStruct((batch, n), jnp.float32),
        scratch_shapes=[pltpu.VMEM((batch, hidden), jnp.bfloat16)],
    )(rows128, offs, W1t, b1.reshape(1, hidden), W2, b2.reshape(1, n))


def kernel(x, embedding, W1, b1, W2, b2):
    num_cat, dim = embedding.shape
    table128 = embedding.reshape(num_cat // _PACK, dim * _PACK)
    x = x.astype(jnp.int32)
    rows128 = _gather_sc(x // _PACK, table128)
    offs = (x % _PACK).reshape(x.shape[0], 1)
    W1t = jnp.tile(W1, (_PACK, 1))
    return _dense(rows128, offs, W1t, b1, W2, b2)


# speedup vs baseline: 1.0049x; 1.0024x over previous
import functools

import jax
import jax.numpy as jnp
from jax import lax
from jax.experimental import pallas as pl
from jax.experimental.pallas import tpu as pltpu
from jax.experimental.pallas import tpu_sc as plsc

_NUM_CORES = 2
_NUM_SUBCORES = 16
_NW = _NUM_CORES * _NUM_SUBCORES

_PACK = 8
_TILE_N = 4096


def _gather_sc(x8, table128):
    batch = x8.shape[0]
    width = table128.shape[1]
    b_per_w = batch // _NW
    mesh = plsc.VectorSubcoreMesh(core_axis_name="c", subcore_axis_name="s")

    @functools.partial(
        pl.kernel,
        mesh=mesh,
        out_type=jax.ShapeDtypeStruct((batch, width), jnp.float32),
        scratch_types=[
            pltpu.VMEM((b_per_w,), jnp.int32),
            pltpu.VMEM((b_per_w, width), jnp.float32),
            pltpu.SemaphoreType.DMA,
        ],
    )
    def gather_kernel(idx_hbm, table_hbm, out_hbm, idx_v, rows_v, sem):
        wid = lax.axis_index("s") * _NUM_CORES + lax.axis_index("c")
        base = wid * b_per_w
        pltpu.sync_copy(idx_hbm.at[pl.ds(base, b_per_w)], idx_v)
        pltpu.async_copy(table_hbm.at[idx_v], rows_v, sem).wait()
        pltpu.sync_copy(rows_v, out_hbm.at[pl.ds(base, b_per_w)])

    return gather_kernel(x8, table128)


def _dense_body(rows_ref, off_ref, w1t_ref, b1_ref, w2_ref, b2_ref, out_ref,
                h_ref):
    @pl.when(pl.program_id(0) == 0)
    def _():
        lane_slot = lax.broadcasted_iota(jnp.int32, rows_ref.shape, 1) // 16
        masked = jnp.where(lane_slot == off_ref[...], rows_ref[...], 0.0)
        h = jnp.dot(masked, w1t_ref[...],
                    preferred_element_type=jnp.float32,
                    precision=lax.Precision.HIGHEST) + b1_ref[...]
        h_ref[...] = jnp.maximum(h, 0.0).astype(jnp.bfloat16)

    out_ref[...] = jnp.dot(h_ref[...], w2_ref[...].astype(jnp.bfloat16),
                           preferred_element_type=jnp.float32) + b2_ref[...]


def _dense(rows128, offs, W1t, b1, W2, b2):
    batch, width = rows128.shape
    hidden = W1t.shape[1]
    n = W2.shape[1]
    grid = pl.cdiv(n, _TILE_N)
    return pl.pallas_call(
        _dense_body,
        grid=(grid,),
        in_specs=[
            pl.BlockSpec((batch, width), lambda i: (0, 0)),
            pl.BlockSpec((batch, 1), lambda i: (0, 0)),
            pl.BlockSpec((width, hidden), lambda i: (0, 0)),
            pl.BlockSpec((1, hidden), lambda i: (0, 0)),
            pl.BlockSpec((W2.shape[0], _TILE_N), lambda i: (0, i)),
            pl.BlockSpec((1, _TILE_N), lambda i: (0, i)),
        ],
        out_specs=pl.BlockSpec((batch, _TILE_N), lambda i: (0, i)),
        out_shape=jax.ShapeDtypeStruct((batch, n), jnp.float32),
        scratch_shapes=[pltpu.VMEM((batch, hidden), jnp.bfloat16)],
    )(rows128, offs, W1t, b1.reshape(1, hidden), W2, b2.reshape(1, n))


def kernel(x, embedding, W1, b1, W2, b2):
    num_cat, dim = embedding.shape
    table128 = embedding.reshape(num_cat // _PACK, dim * _PACK)
    x = x.astype(jnp.int32)
    rows128 = _gather_sc(x // _PACK, table128)
    offs = (x % _PACK).reshape(x.shape[0], 1)
    W1t = jnp.tile(W1, (_PACK, 1))
    return _dense(rows128, offs, W1t, b1, W2, b2)


# speedup vs baseline: 1.0063x; 1.0014x over previous
import functools

import jax
import jax.numpy as jnp
from jax import lax
from jax.experimental import pallas as pl
from jax.experimental.pallas import tpu as pltpu
from jax.experimental.pallas import tpu_sc as plsc

_NUM_CORES = 2
_NUM_SUBCORES = 16
_NW = _NUM_CORES * _NUM_SUBCORES

_PACK = 8
_TILE_N = 1024


def _gather_sc(x8, table128):
    batch = x8.shape[0]
    width = table128.shape[1]
    b_per_w = batch // _NW
    mesh = plsc.VectorSubcoreMesh(core_axis_name="c", subcore_axis_name="s")

    @functools.partial(
        pl.kernel,
        mesh=mesh,
        out_type=jax.ShapeDtypeStruct((batch, width), jnp.float32),
        scratch_types=[
            pltpu.VMEM((b_per_w,), jnp.int32),
            pltpu.VMEM((b_per_w, width), jnp.float32),
            pltpu.SemaphoreType.DMA,
        ],
    )
    def gather_kernel(idx_hbm, table_hbm, out_hbm, idx_v, rows_v, sem):
        wid = lax.axis_index("s") * _NUM_CORES + lax.axis_index("c")
        base = wid * b_per_w
        pltpu.sync_copy(idx_hbm.at[pl.ds(base, b_per_w)], idx_v)
        pltpu.async_copy(table_hbm.at[idx_v], rows_v, sem).wait()
        pltpu.sync_copy(rows_v, out_hbm.at[pl.ds(base, b_per_w)])

    return gather_kernel(x8, table128)


_NBUF = 8


def _mlp_head(rows_ref, off_ref, w1t_ref, b1_ref):
    lane_slot = lax.broadcasted_iota(jnp.int32, rows_ref.shape, 1) // 16
    masked = jnp.where(lane_slot == off_ref[...], rows_ref[...], 0.0)
    h = jnp.dot(masked, w1t_ref[...],
                preferred_element_type=jnp.float32,
                precision=lax.Precision.HIGHEST) + b1_ref[...]
    return jnp.maximum(h, 0.0).astype(jnp.bfloat16)


def _dense(rows128, offs, W1t, b1, W2, b2):
    batch, width = rows128.shape
    hidden = W1t.shape[1]
    n = W2.shape[1]
    n_aligned = (n // 128) * 128
    nt = pl.cdiv(n_aligned, _TILE_N)
    tail = n_aligned - (nt - 1) * _TILE_N

    def body(rows_ref, off_ref, w1t_ref, b1_ref, w2_ref, b2_ref, out_hbm,
             h_ref, obuf, sems):
        i = pl.program_id(0)
        b = lax.rem(i, _NBUF)

        @pl.when(i == 0)
        def _():
            h_ref[...] = _mlp_head(rows_ref, off_ref, w1t_ref, b1_ref)

        @pl.when(i >= _NBUF)
        def _():
            j = pl.multiple_of((i - _NBUF) * _TILE_N, 128)
            pltpu.make_async_copy(
                obuf.at[b],
                out_hbm.at[:, pl.ds(j, _TILE_N)],
                sems.at[b],
            ).wait()

        obuf[b] = jnp.dot(h_ref[...], w2_ref[...].astype(jnp.bfloat16),
                          preferred_element_type=jnp.float32) + b2_ref[...]

        @pl.when(i < nt - 1)
        def _():
            pltpu.make_async_copy(
                obuf.at[b],
                out_hbm.at[:, pl.ds(pl.multiple_of(i * _TILE_N, 128), _TILE_N)],
                sems.at[b],
            ).start()

        @pl.when(i == nt - 1)
        def _():
            pltpu.make_async_copy(
                obuf.at[b, :, 0:tail],
                out_hbm.at[:, pl.ds((nt - 1) * _TILE_N, tail)],
                sems.at[b],
            ).start()
            for k in range(nt - _NBUF, nt):
                kb = k % _NBUF
                if k < nt - 1:
                    pltpu.make_async_copy(
                        obuf.at[kb],
                        out_hbm.at[:, pl.ds(k * _TILE_N, _TILE_N)],
                        sems.at[kb],
                    ).wait()
                else:
                    pltpu.make_async_copy(
                        obuf.at[kb, :, 0:tail],
                        out_hbm.at[:, pl.ds(k * _TILE_N, tail)],
                        sems.at[kb],
                    ).wait()

    out_main = pl.pallas_call(
        body,
        grid=(nt,),
        in_specs=[
            pl.BlockSpec((batch, width), lambda i: (0, 0)),
            pl.BlockSpec((batch, 1), lambda i: (0, 0)),
            pl.BlockSpec((width, hidden), lambda i: (0, 0)),
            pl.BlockSpec((1, hidden), lambda i: (0, 0)),
            pl.BlockSpec((W2.shape[0], _TILE_N), lambda i: (0, i)),
            pl.BlockSpec((1, _TILE_N), lambda i: (0, i)),
        ],
        out_specs=pl.BlockSpec(memory_space=pltpu.MemorySpace.HBM),
        out_shape=jax.ShapeDtypeStruct((batch, n), jnp.float32),
        scratch_shapes=[
            pltpu.VMEM((batch, hidden), jnp.bfloat16),
            pltpu.VMEM((_NBUF, batch, _TILE_N), jnp.float32),
            pltpu.SemaphoreType.DMA((_NBUF,)),
        ],
    )(rows128, offs, W1t, b1.reshape(1, hidden), W2, b2.reshape(1, n))

    lastblk = n_aligned // 128

    def tail_body(rows_ref, off_ref, w1t_ref, b1_ref, w2_ref, b2_ref,
                  alias_ref, out_ref):
        h = _mlp_head(rows_ref, off_ref, w1t_ref, b1_ref)
        out_ref[...] = jnp.dot(h, w2_ref[...].astype(jnp.bfloat16),
                               preferred_element_type=jnp.float32) + b2_ref[...]

    return pl.pallas_call(
        tail_body,
        grid=(1,),
        in_specs=[
            pl.BlockSpec((batch, width), lambda i: (0, 0)),
            pl.BlockSpec((batch, 1), lambda i: (0, 0)),
            pl.BlockSpec((width, hidden), lambda i: (0, 0)),
            pl.BlockSpec((1, hidden), lambda i: (0, 0)),
            pl.BlockSpec((W2.shape[0], 128), lambda i: (0, lastblk)),
            pl.BlockSpec((1, 128), lambda i: (0, lastblk)),
            pl.BlockSpec(memory_space=pltpu.MemorySpace.HBM),
        ],
        out_specs=pl.BlockSpec((batch, 128), lambda i: (0, lastblk)),
        out_shape=jax.ShapeDtypeStruct((batch, n), jnp.float32),
        input_output_aliases={6: 0},
    )(rows128, offs, W1t, b1.reshape(1, hidden), W2, b2.reshape(1, n),
      out_main)


def kernel(x, embedding, W1, b1, W2, b2):
    num_cat, dim = embedding.shape
    table128 = embedding.reshape(num_cat // _PACK, dim * _PACK)
    x = x.astype(jnp.int32)
    rows128 = _gather_sc(x // _PACK, table128)
    offs = (x % _PACK).reshape(x.shape[0], 1)
    W1t = jnp.tile(W1, (_PACK, 1))
    return _dense(rows128, offs, W1t, b1, W2, b2)
